# baseline (speedup 1.0000x reference)
import jax
import jax.numpy as jnp
from jax import lax
from jax.experimental import pallas as pl
from jax.experimental.pallas import tpu as pltpu
from jax.experimental.pallas import tpu_sc as plsc

B, MAX_LEN, L, EMB, RNN, ATT, HID, VOCAB = 32, 50, 32, 64, 128, 128, 256, 625
S = B * MAX_LEN
SIDE_ROWS = S * L
D2 = 2 * RNN

_NC, _NS = 2, 16
_NW = _NC * _NS
_ROWS_PER_TILE = 2 * SIDE_ROWS // _NW
_CHUNK = 128
_NCHUNKS = _ROWS_PER_TILE // _CHUNK

R = 400
NCH = S // R


def _sc_gather_body(table_hbm, idx_hbm, out_hbm, idx_v,
                    rows0, rows1, gs0, gs1, ws0, ws1):
    wid = lax.axis_index("s") * _NC + lax.axis_index("c")
    pltpu.sync_copy(idx_hbm.at[wid], idx_v)
    base = wid * _ROWS_PER_TILE
    rows = [rows0, rows1]
    gs = [gs0, gs1]
    ws = [ws0, ws1]
    gcopy = [None, None]
    wcopy = [None, None]
    gcopy[0] = pltpu.async_copy(table_hbm.at[idx_v.at[0]], rows0, gs0)
    for j in range(_NCHUNKS):
        cur = j % 2
        nxt = 1 - cur
        if j + 1 < _NCHUNKS:
            if wcopy[nxt] is not None:
                wcopy[nxt].wait()
            gcopy[nxt] = pltpu.async_copy(
                table_hbm.at[idx_v.at[j + 1]], rows[nxt], gs[nxt])
        gcopy[cur].wait()
        wcopy[cur] = pltpu.async_copy(
            rows[cur], out_hbm.at[pl.ds(base + j * _CHUNK, _CHUNK)], ws[cur])
    wcopy[0].wait()
    wcopy[1].wait()


def _sc_gather(table, idx3d):
    mesh = plsc.VectorSubcoreMesh(core_axis_name="c", subcore_axis_name="s")
    fn = pl.kernel(
        _sc_gather_body,
        out_type=jax.ShapeDtypeStruct((2 * SIDE_ROWS, EMB), jnp.float32),
        mesh=mesh,
        scratch_types=[
            pltpu.VMEM((_NCHUNKS, _CHUNK), jnp.int32),
            pltpu.VMEM((_CHUNK, EMB), jnp.float32),
            pltpu.VMEM((_CHUNK, EMB), jnp.float32),
            pltpu.SemaphoreType.DMA,
            pltpu.SemaphoreType.DMA,
            pltpu.SemaphoreType.DMA,
            pltpu.SemaphoreType.DMA,
        ],
        compiler_params=pltpu.CompilerParams(use_tc_tiling_on_sc=False),
    )
    return fn(table, idx3d)


def _k1_body(g_ref, kf_ref, kb_ref, rkf_ref, rkb_ref, bi_ref,
             sw_ref, sbw_ref, sv_ref, out_ref, ra_ref):
    kf = kf_ref[...]
    kb = kb_ref[...]
    rkf = rkf_ref[...]
    rkb = rkb_ref[...]
    bif = bi_ref[0:1, :]
    brf = bi_ref[1:2, :]
    bib = bi_ref[2:3, :]
    brb = bi_ref[3:4, :]

    def step(i, carry):
        hf, hb = carry
        tb = L - 1 - i
        xf = g_ref[i]
        xb = g_ref[tb]
        zxf = jnp.dot(xf, kf, preferred_element_type=jnp.float32) + bif
        zhf = jnp.dot(hf, rkf, preferred_element_type=jnp.float32) + brf
        zxb = jnp.dot(xb, kb, preferred_element_type=jnp.float32) + bib
        zhb = jnp.dot(hb, rkb, preferred_element_type=jnp.float32) + brb
        zf = jax.nn.sigmoid(zxf[:, :RNN] + zhf[:, :RNN])
        rf = jax.nn.sigmoid(zxf[:, RNN:2 * RNN] + zhf[:, RNN:2 * RNN])
        cf = jnp.tanh(zxf[:, 2 * RNN:] + rf * zhf[:, 2 * RNN:])
        hf = cf + zf * (hf - cf)
        zb = jax.nn.sigmoid(zxb[:, :RNN] + zhb[:, :RNN])
        rb = jax.nn.sigmoid(zxb[:, RNN:2 * RNN] + zhb[:, RNN:2 * RNN])
        cb = jnp.tanh(zxb[:, 2 * RNN:] + rb * zhb[:, 2 * RNN:])
        hb = cb + zb * (hb - cb)
        ra_ref[i, :, :RNN] = hf
        ra_ref[tb, :, RNN:] = hb
        return hf, hb

    h0 = jnp.zeros((R, RNN), jnp.float32)
    lax.fori_loop(0, L, step, (h0, h0))

    ra = ra_ref[...]
    u = jnp.tanh(
        jnp.dot(ra.reshape(L * R, D2), sw_ref[...],
                preferred_element_type=jnp.float32) + sbw_ref[...])
    sc = jnp.sum(u.reshape(L, R, ATT) * sv_ref[...], axis=2)
    sc = sc - jnp.max(sc, axis=0, keepdims=True)
    e = jnp.exp(sc)
    w = e / jnp.sum(e, axis=0, keepdims=True)
    out_ref[...] = jnp.sum(w[:, :, None] * ra, axis=0)


def _k2_body(va_ref, vd_ref, la_ref, ld_ref,
             waw_ref, wabw_ref, wav_ref, wdw_ref, wdbw_ref, wdv_ref,
             fdw_ref, fdb_ref, prw_ref, prb_ref, out_ref,
             ma_ref, md_ref, ad_ref, da_ref):
    iota = lax.broadcasted_iota(jnp.int32, (B, MAX_LEN), 1)
    ma_ref[...] = (iota < la_ref[...]).astype(jnp.float32)
    md_ref[...] = (iota < ld_ref[...]).astype(jnp.float32)

    def batch_step(b, carry):
        va_b = va_ref[b]
        vd_b = vd_ref[b]
        ma = ma_ref[b]
        md = md_ref[b]
        s1 = lax.dot_general(vd_b, va_b, (((1,), (1,)), ((), ())),
                             preferred_element_type=jnp.float32)
        s1 = s1 + (1.0 - ma) * (-1e9)
        e1 = jnp.exp(s1 - jnp.max(s1, axis=1, keepdims=True))
        w1 = e1 / jnp.sum(e1, axis=1, keepdims=True)
        ad_ref[b] = jnp.dot(w1, va_b, preferred_element_type=jnp.float32)
        s2 = lax.dot_general(va_b, vd_b, (((1,), (1,)), ((), ())),
                             preferred_element_type=jnp.float32)
        s2 = s2 + (1.0 - md) * (-1e9)
        e2 = jnp.exp(s2 - jnp.max(s2, axis=1, keepdims=True))
        w2 = e2 / jnp.sum(e2, axis=1, keepdims=True)
        da_ref[b] = jnp.dot(w2, vd_b, preferred_element_type=jnp.float32)
        return carry

    lax.fori_loop(0, B, batch_step, 0)

    vabyd = ad_ref[...] * md_ref[...][:, :, None]
    vdbya = da_ref[...] * ma_ref[...][:, :, None]

    def final_att(vals, w_ref, bw_ref, v_ref):
        u = jnp.tanh(
            jnp.dot(vals.reshape(B * MAX_LEN, D2), w_ref[...],
                    preferred_element_type=jnp.float32) + bw_ref[...])
        sc = jnp.sum(u.reshape(B, MAX_LEN, ATT) * v_ref[...], axis=2)
        sc = sc - jnp.max(sc, axis=1, keepdims=True)
        e = jnp.exp(sc)
        w = e / jnp.sum(e, axis=1, keepdims=True)
        return jnp.sum(w[:, :, None] * vals, axis=1)

    ca = final_att(vabyd, waw_ref, wabw_ref, wav_ref)
    cd = final_att(vdbya, wdw_ref, wdbw_ref, wdv_ref)
    feat = jnp.concatenate([ca, cd], axis=1)
    h = jnp.dot(feat, fdw_ref[...], preferred_element_type=jnp.float32) \
        + fdb_ref[...]
    h = jnp.where(h > 0, h, jnp.exp(h) - 1.0)
    o = jnp.sum(h * prw_ref[...], axis=1, keepdims=True) + prb_ref[...]
    out_ref[...] = jax.nn.sigmoid(o)


def _k1_call(G, side, Kf, Kb, RKf, RKb, BI, sW, sbW, sV, interpret=False):
    full = lambda c: (0, 0)
    return pl.pallas_call(
        _k1_body,
        grid=(NCH,),
        in_specs=[
            pl.BlockSpec((L, R, EMB), lambda c: (side, c, 0)),
            pl.BlockSpec((EMB, 3 * RNN), full),
            pl.BlockSpec((EMB, 3 * RNN), full),
            pl.BlockSpec((RNN, 3 * RNN), full),
            pl.BlockSpec((RNN, 3 * RNN), full),
            pl.BlockSpec((4, 3 * RNN), full),
            pl.BlockSpec((D2, ATT), full),
            pl.BlockSpec((1, ATT), full),
            pl.BlockSpec((1, ATT), full),
        ],
        out_specs=pl.BlockSpec((R, D2), lambda c: (c, 0)),
        out_shape=jax.ShapeDtypeStruct((S, D2), jnp.float32),
        scratch_shapes=[pltpu.VMEM((L, R, D2), jnp.float32)],
        compiler_params=pltpu.CompilerParams(
            dimension_semantics=("parallel",),
            vmem_limit_bytes=100 * 1024 * 1024),
        interpret=interpret,
    )(G, Kf, Kb, RKf, RKb, BI, sW, sbW, sV)


def _k2_call(va, vd, la, ld, waW, wabW, waV, wdW, wdbW, wdV,
             fdW, fdb, prW, prb, interpret=False):
    return pl.pallas_call(
        _k2_body,
        out_shape=jax.ShapeDtypeStruct((B, 1), jnp.float32),
        scratch_shapes=[
            pltpu.VMEM((B, MAX_LEN), jnp.float32),
            pltpu.VMEM((B, MAX_LEN), jnp.float32),
            pltpu.VMEM((B, MAX_LEN, D2), jnp.float32),
            pltpu.VMEM((B, MAX_LEN, D2), jnp.float32),
        ],
        interpret=interpret,
    )(va, vd, la, ld, waW, wabW, waV, wdW, wdbW, wdV, fdW, fdb, prW, prb)


def kernel(xa, xd, xlen_a, xlen_d, emb_table,
           gru_af_k, gru_af_rk, gru_af_b, gru_ab_k, gru_ab_rk, gru_ab_b,
           gru_df_k, gru_df_rk, gru_df_b, gru_db_k, gru_db_rk, gru_db_b,
           satt_a_W, satt_a_bW, satt_a_V, satt_a_bV,
           satt_d_W, satt_d_bW, satt_d_V, satt_d_bV,
           fatt_a_W, fatt_a_bW, fatt_a_V, fatt_a_bV,
           fatt_d_W, fatt_d_bW, fatt_d_V, fatt_d_bV,
           fd_W, fd_b, pr_W, pr_b):
    table = emb_table.astype(jnp.float32)
    idx_a = jnp.transpose(xa.reshape(S, L)).astype(jnp.int32).reshape(-1)
    idx_d = jnp.transpose(xd.reshape(S, L)).astype(jnp.int32).reshape(-1)
    idx_all = jnp.concatenate([idx_a, idx_d]).reshape(_NW, _NCHUNKS, _CHUNK)

    G = _sc_gather(table, idx_all).reshape(2 * L, S, EMB)

    BIa = jnp.stack([gru_af_b[0], gru_af_b[1], gru_ab_b[0], gru_ab_b[1]])
    BId = jnp.stack([gru_df_b[0], gru_df_b[1], gru_db_b[0], gru_db_b[1]])

    va = _k1_call(G, 0, gru_af_k, gru_ab_k, gru_af_rk, gru_ab_rk, BIa,
                  satt_a_W, satt_a_bW.reshape(1, ATT),
                  satt_a_V.reshape(1, ATT)).reshape(B, MAX_LEN, D2)
    vd = _k1_call(G, 1, gru_df_k, gru_db_k, gru_df_rk, gru_db_rk, BId,
                  satt_d_W, satt_d_bW.reshape(1, ATT),
                  satt_d_V.reshape(1, ATT)).reshape(B, MAX_LEN, D2)

    out = _k2_call(
        va, vd,
        xlen_a.astype(jnp.int32).reshape(B, 1),
        xlen_d.astype(jnp.int32).reshape(B, 1),
        fatt_a_W, fatt_a_bW.reshape(1, ATT), fatt_a_V.reshape(1, ATT),
        fatt_d_W, fatt_d_bW.reshape(1, ATT), fatt_d_V.reshape(1, ATT),
        fd_W, fd_b.reshape(1, HID), pr_W.reshape(1, HID),
        pr_b.reshape(1, 1))
    return out


# speedup vs baseline: 1.0322x; 1.0322x over previous
import jax
import jax.numpy as jnp
from jax import lax
from jax.experimental import pallas as pl
from jax.experimental.pallas import tpu as pltpu
from jax.experimental.pallas import tpu_sc as plsc

B, MAX_LEN, L, EMB, RNN, ATT, HID, VOCAB = 32, 50, 32, 64, 128, 128, 256, 625
S = B * MAX_LEN
SIDE_ROWS = S * L
D2 = 2 * RNN

_NC, _NS = 2, 16
_NW = _NC * _NS
_ROWS_PER_TILE = SIDE_ROWS // _NW
_CHUNK = 80
_NCHUNKS = _ROWS_PER_TILE // _CHUNK

R = 400
NCH = S // R


def _sc_gather_body(table_hbm, idx_hbm, out_hbm, idx_v,
                    rows0, rows1, gs0, gs1, ws0, ws1):
    wid = lax.axis_index("s") * _NC + lax.axis_index("c")
    pltpu.sync_copy(idx_hbm.at[wid], idx_v)
    base = wid * _ROWS_PER_TILE
    rows = [rows0, rows1]
    gs = [gs0, gs1]
    ws = [ws0, ws1]
    gcopy = [None, None]
    wcopy = [None, None]
    gcopy[0] = pltpu.async_copy(table_hbm.at[idx_v.at[0]], rows0, gs0)
    for j in range(_NCHUNKS):
        cur = j % 2
        nxt = 1 - cur
        if j + 1 < _NCHUNKS:
            if wcopy[nxt] is not None:
                wcopy[nxt].wait()
            gcopy[nxt] = pltpu.async_copy(
                table_hbm.at[idx_v.at[j + 1]], rows[nxt], gs[nxt])
        gcopy[cur].wait()
        wcopy[cur] = pltpu.async_copy(
            rows[cur], out_hbm.at[pl.ds(base + j * _CHUNK, _CHUNK)], ws[cur])
    wcopy[0].wait()
    wcopy[1].wait()


def _sc_gather(table, idx3d):
    mesh = plsc.VectorSubcoreMesh(core_axis_name="c", subcore_axis_name="s")
    fn = pl.kernel(
        _sc_gather_body,
        out_type=jax.ShapeDtypeStruct((SIDE_ROWS, EMB), jnp.float32),
        mesh=mesh,
        scratch_types=[
            pltpu.VMEM((_NCHUNKS, _CHUNK), jnp.int32),
            pltpu.VMEM((_CHUNK, EMB), jnp.float32),
            pltpu.VMEM((_CHUNK, EMB), jnp.float32),
            pltpu.SemaphoreType.DMA,
            pltpu.SemaphoreType.DMA,
            pltpu.SemaphoreType.DMA,
            pltpu.SemaphoreType.DMA,
        ],
        compiler_params=pltpu.CompilerParams(use_tc_tiling_on_sc=False),
    )
    return fn(table, idx3d)


def _k1_body(g_ref, kf_ref, kb_ref, rkf_ref, rkb_ref, bi_ref,
             sw_ref, sbw_ref, sv_ref, out_ref, ra_ref):
    kf = kf_ref[...]
    kb = kb_ref[...]
    rkf = rkf_ref[...]
    rkb = rkb_ref[...]
    bif = bi_ref[0:1, :]
    brf = bi_ref[1:2, :]
    bib = bi_ref[2:3, :]
    brb = bi_ref[3:4, :]

    def step(i, carry):
        hf, hb = carry
        tb = L - 1 - i
        xf = g_ref[i]
        xb = g_ref[tb]
        zxf = jnp.dot(xf, kf, preferred_element_type=jnp.float32) + bif
        zhf = jnp.dot(hf, rkf, preferred_element_type=jnp.float32) + brf
        zxb = jnp.dot(xb, kb, preferred_element_type=jnp.float32) + bib
        zhb = jnp.dot(hb, rkb, preferred_element_type=jnp.float32) + brb
        zf = jax.nn.sigmoid(zxf[:, :RNN] + zhf[:, :RNN])
        rf = jax.nn.sigmoid(zxf[:, RNN:2 * RNN] + zhf[:, RNN:2 * RNN])
        cf = jnp.tanh(zxf[:, 2 * RNN:] + rf * zhf[:, 2 * RNN:])
        hf = cf + zf * (hf - cf)
        zb = jax.nn.sigmoid(zxb[:, :RNN] + zhb[:, :RNN])
        rb = jax.nn.sigmoid(zxb[:, RNN:2 * RNN] + zhb[:, RNN:2 * RNN])
        cb = jnp.tanh(zxb[:, 2 * RNN:] + rb * zhb[:, 2 * RNN:])
        hb = cb + zb * (hb - cb)
        ra_ref[i, :, :RNN] = hf
        ra_ref[tb, :, RNN:] = hb
        return hf, hb

    h0 = jnp.zeros((R, RNN), jnp.float32)
    lax.fori_loop(0, L, step, (h0, h0))

    ra = ra_ref[...]
    u = jnp.tanh(
        jnp.dot(ra.reshape(L * R, D2), sw_ref[...],
                preferred_element_type=jnp.float32) + sbw_ref[...])
    sc = jnp.sum(u.reshape(L, R, ATT) * sv_ref[...], axis=2)
    sc = sc - jnp.max(sc, axis=0, keepdims=True)
    e = jnp.exp(sc)
    w = e / jnp.sum(e, axis=0, keepdims=True)
    out_ref[...] = jnp.sum(w[:, :, None] * ra, axis=0)


def _k2_body(va_ref, vd_ref, la_ref, ld_ref,
             waw_ref, wabw_ref, wav_ref, wdw_ref, wdbw_ref, wdv_ref,
             fdw_ref, fdb_ref, prw_ref, prb_ref, out_ref,
             ma_ref, md_ref, ad_ref, da_ref):
    iota = lax.broadcasted_iota(jnp.int32, (B, MAX_LEN), 1)
    ma_ref[...] = (iota < la_ref[...]).astype(jnp.float32)
    md_ref[...] = (iota < ld_ref[...]).astype(jnp.float32)

    def batch_step(b, carry):
        va_b = va_ref[b]
        vd_b = vd_ref[b]
        ma = ma_ref[b]
        md = md_ref[b]
        s1 = lax.dot_general(vd_b, va_b, (((1,), (1,)), ((), ())),
                             preferred_element_type=jnp.float32)
        s1 = s1 + (1.0 - ma) * (-1e9)
        e1 = jnp.exp(s1 - jnp.max(s1, axis=1, keepdims=True))
        w1 = e1 / jnp.sum(e1, axis=1, keepdims=True)
        ad_ref[b] = jnp.dot(w1, va_b, preferred_element_type=jnp.float32)
        s2 = lax.dot_general(va_b, vd_b, (((1,), (1,)), ((), ())),
                             preferred_element_type=jnp.float32)
        s2 = s2 + (1.0 - md) * (-1e9)
        e2 = jnp.exp(s2 - jnp.max(s2, axis=1, keepdims=True))
        w2 = e2 / jnp.sum(e2, axis=1, keepdims=True)
        da_ref[b] = jnp.dot(w2, vd_b, preferred_element_type=jnp.float32)
        return carry

    lax.fori_loop(0, B, batch_step, 0)

    vabyd = ad_ref[...] * md_ref[...][:, :, None]
    vdbya = da_ref[...] * ma_ref[...][:, :, None]

    def final_att(vals, w_ref, bw_ref, v_ref):
        u = jnp.tanh(
            jnp.dot(vals.reshape(B * MAX_LEN, D2), w_ref[...],
                    preferred_element_type=jnp.float32) + bw_ref[...])
        sc = jnp.sum(u.reshape(B, MAX_LEN, ATT) * v_ref[...], axis=2)
        sc = sc - jnp.max(sc, axis=1, keepdims=True)
        e = jnp.exp(sc)
        w = e / jnp.sum(e, axis=1, keepdims=True)
        return jnp.sum(w[:, :, None] * vals, axis=1)

    ca = final_att(vabyd, waw_ref, wabw_ref, wav_ref)
    cd = final_att(vdbya, wdw_ref, wdbw_ref, wdv_ref)
    feat = jnp.concatenate([ca, cd], axis=1)
    h = jnp.dot(feat, fdw_ref[...], preferred_element_type=jnp.float32) \
        + fdb_ref[...]
    h = jnp.where(h > 0, h, jnp.exp(h) - 1.0)
    o = jnp.sum(h * prw_ref[...], axis=1, keepdims=True) + prb_ref[...]
    out_ref[...] = jax.nn.sigmoid(o)


def _k1_call(G, Kf, Kb, RKf, RKb, BI, sW, sbW, sV, interpret=False):
    full = lambda c: (0, 0)
    return pl.pallas_call(
        _k1_body,
        grid=(NCH,),
        in_specs=[
            pl.BlockSpec((L, R, EMB), lambda c: (0, c, 0)),
            pl.BlockSpec((EMB, 3 * RNN), full),
            pl.BlockSpec((EMB, 3 * RNN), full),
            pl.BlockSpec((RNN, 3 * RNN), full),
            pl.BlockSpec((RNN, 3 * RNN), full),
            pl.BlockSpec((4, 3 * RNN), full),
            pl.BlockSpec((D2, ATT), full),
            pl.BlockSpec((1, ATT), full),
            pl.BlockSpec((1, ATT), full),
        ],
        out_specs=pl.BlockSpec((R, D2), lambda c: (c, 0)),
        out_shape=jax.ShapeDtypeStruct((S, D2), jnp.float32),
        scratch_shapes=[pltpu.VMEM((L, R, D2), jnp.float32)],
        compiler_params=pltpu.CompilerParams(
            dimension_semantics=("parallel",),
            vmem_limit_bytes=100 * 1024 * 1024),
        interpret=interpret,
    )(G, Kf, Kb, RKf, RKb, BI, sW, sbW, sV)


def _k2_call(va, vd, la, ld, waW, wabW, waV, wdW, wdbW, wdV,
             fdW, fdb, prW, prb, interpret=False):
    return pl.pallas_call(
        _k2_body,
        out_shape=jax.ShapeDtypeStruct((B, 1), jnp.float32),
        scratch_shapes=[
            pltpu.VMEM((B, MAX_LEN), jnp.float32),
            pltpu.VMEM((B, MAX_LEN), jnp.float32),
            pltpu.VMEM((B, MAX_LEN, D2), jnp.float32),
            pltpu.VMEM((B, MAX_LEN, D2), jnp.float32),
        ],
        interpret=interpret,
    )(va, vd, la, ld, waW, wabW, waV, wdW, wdbW, wdV, fdW, fdb, prW, prb)


def kernel(xa, xd, xlen_a, xlen_d, emb_table,
           gru_af_k, gru_af_rk, gru_af_b, gru_ab_k, gru_ab_rk, gru_ab_b,
           gru_df_k, gru_df_rk, gru_df_b, gru_db_k, gru_db_rk, gru_db_b,
           satt_a_W, satt_a_bW, satt_a_V, satt_a_bV,
           satt_d_W, satt_d_bW, satt_d_V, satt_d_bV,
           fatt_a_W, fatt_a_bW, fatt_a_V, fatt_a_bV,
           fatt_d_W, fatt_d_bW, fatt_d_V, fatt_d_bV,
           fd_W, fd_b, pr_W, pr_b):
    table = emb_table.astype(jnp.float32)
    idx_a = jnp.transpose(xa.reshape(S, L)).astype(jnp.int32).reshape(
        _NW, _NCHUNKS, _CHUNK)
    idx_d = jnp.transpose(xd.reshape(S, L)).astype(jnp.int32).reshape(
        _NW, _NCHUNKS, _CHUNK)

    Ga = _sc_gather(table, idx_a).reshape(L, S, EMB)
    Gd = _sc_gather(table, idx_d).reshape(L, S, EMB)

    BIa = jnp.stack([gru_af_b[0], gru_af_b[1], gru_ab_b[0], gru_ab_b[1]])
    BId = jnp.stack([gru_df_b[0], gru_df_b[1], gru_db_b[0], gru_db_b[1]])

    va = _k1_call(Ga, gru_af_k, gru_ab_k, gru_af_rk, gru_ab_rk, BIa,
                  satt_a_W, satt_a_bW.reshape(1, ATT),
                  satt_a_V.reshape(1, ATT)).reshape(B, MAX_LEN, D2)
    vd = _k1_call(Gd, gru_df_k, gru_db_k, gru_df_rk, gru_db_rk, BId,
                  satt_d_W, satt_d_bW.reshape(1, ATT),
                  satt_d_V.reshape(1, ATT)).reshape(B, MAX_LEN, D2)

    out = _k2_call(
        va, vd,
        xlen_a.astype(jnp.int32).reshape(B, 1),
        xlen_d.astype(jnp.int32).reshape(B, 1),
        fatt_a_W, fatt_a_bW.reshape(1, ATT), fatt_a_V.reshape(1, ATT),
        fatt_d_W, fatt_d_bW.reshape(1, ATT), fatt_d_V.reshape(1, ATT),
        fd_W, fd_b.reshape(1, HID), pr_W.reshape(1, HID),
        pr_b.reshape(1, 1))
    return out
